# baseline (speedup 1.0000x reference)
import functools
import math

import jax
import jax.numpy as jnp
from jax import lax
from jax.experimental import pallas as pl
from jax.experimental.pallas import tpu as pltpu
from jax.experimental.pallas import tpu_sc as plsc

D_MODEL = 1024
NUM_WORKERS = 32
CHUNK = 32
NBUF = 3
SC_ROWS = 19456
TC_BLK = 512

_SIN_COEF = (6.2831854820251465, -41.341697692871094, 81.60502624511719,
             -76.7015380859375, 42.016075134277344, -14.868322372436523,
             3.199338912963867)
_MAGIC = 1.5 * 2.0 ** 23


def _gather_body(b_per_w, n_chunks, ids_hbm, table_hbm, out_hbm,
                 idx_v, rows_v, gsems, ssems):
    nc = 2
    wid = lax.axis_index("s") * nc + lax.axis_index("c")
    base = wid * b_per_w

    pltpu.sync_copy(ids_hbm.at[pl.ds(base, b_per_w)], idx_v)

    def gather(c, buf):
        return pltpu.make_async_copy(
            table_hbm.at[idx_v.at[pl.ds(c * CHUNK, CHUNK)]],
            rows_v.at[buf],
            gsems.at[buf],
        )

    def scatter(c, buf):
        return pltpu.make_async_copy(
            rows_v.at[buf],
            out_hbm.at[pl.ds(base + c * CHUNK, CHUNK)],
            ssems.at[buf],
        )

    for k in range(NBUF):
        gather(k, k).start()

    def body(c, carry):
        buf = lax.rem(c, NBUF)

        @pl.when(c >= 1)
        def _():
            pbuf = lax.rem(c - 1, NBUF)
            scatter(c - 1, pbuf).wait()

            @pl.when(c - 1 + NBUF < n_chunks)
            def _():
                gather(c - 1 + NBUF, pbuf).start()

        gather(c, buf).wait()
        scatter(c, buf).start()
        return carry

    lax.fori_loop(0, n_chunks, body, 0)
    scatter(n_chunks - 1, lax.rem(n_chunks - 1, NBUF)).wait()


def _sin_body(pos_ref, c_ref, q_ref, o_ref):
    t = pos_ref[...] * c_ref[...] + q_ref[...]
    magic = jnp.float32(_MAGIC)
    n = (t + magic) - magic
    r = t - n
    s = r * r
    acc = jnp.float32(_SIN_COEF[-1])
    for a in _SIN_COEF[-2::-1]:
        acc = acc * s + jnp.float32(a)
    o_ref[...] = acc * r


def kernel(position_ids, positional_encoding):
    batch, seq = position_ids.shape
    n_rows = batch * seq
    ids = position_ids.reshape(n_rows).astype(jnp.int32)

    b_per_w = SC_ROWS // NUM_WORKERS
    n_chunks = b_per_w // CHUNK
    mesh = plsc.VectorSubcoreMesh(core_axis_name="c", subcore_axis_name="s")
    body = functools.partial(_gather_body, b_per_w, n_chunks)
    sc_out = pl.kernel(
        body,
        out_type=jax.ShapeDtypeStruct((SC_ROWS, D_MODEL), jnp.float32),
        mesh=mesh,
        scratch_types=[
            pltpu.VMEM((b_per_w,), jnp.int32),
            pltpu.VMEM((NBUF, CHUNK, D_MODEL), jnp.float32),
            pltpu.SemaphoreType.DMA((NBUF,)),
            pltpu.SemaphoreType.DMA((NBUF,)),
        ],
    )(ids, positional_encoding)

    tc_rows = n_rows - SC_ROWS
    pos_tc = ids[SC_ROWS:].reshape(tc_rows, 1).astype(jnp.float32)
    row1 = positional_encoding[1]
    d = jnp.arctan2(row1[0::2], row1[1::2])
    c = (jnp.repeat(d, 2) / (2.0 * math.pi)).reshape(1, D_MODEL)
    q = jnp.tile(jnp.array([0.0, 0.25], jnp.float32),
                 D_MODEL // 2).reshape(1, D_MODEL)
    tc_out = pl.pallas_call(
        _sin_body,
        grid=(tc_rows // TC_BLK,),
        in_specs=[pl.BlockSpec((TC_BLK, 1), lambda i: (i, 0)),
                  pl.BlockSpec((1, D_MODEL), lambda i: (0, 0)),
                  pl.BlockSpec((1, D_MODEL), lambda i: (0, 0))],
        out_specs=pl.BlockSpec((TC_BLK, D_MODEL), lambda i: (i, 0)),
        out_shape=jax.ShapeDtypeStruct((tc_rows, D_MODEL), jnp.float32),
    )(pos_tc, c, q)

    out = jnp.concatenate([sc_out, tc_out], axis=0)
    return out.reshape(batch, seq, D_MODEL)


# speedup vs baseline: 1.8149x; 1.8149x over previous
import functools

import jax
import jax.numpy as jnp
from jax import lax
from jax.experimental import pallas as pl
from jax.experimental.pallas import tpu as pltpu
from jax.experimental.pallas import tpu_sc as plsc

D_MODEL = 1024
NUM_WORKERS = 32
CHUNK = 32
NBUF = 3


def _gather_body(b_per_w, n_chunks, segs_per_batch, ids_hbm, table_hbm,
                 out_hbm, idx_v, rows_v, gsems, ssems):
    nc = 2
    wid = lax.axis_index("s") * nc + lax.axis_index("c")
    batch = wid // segs_per_batch
    seg = lax.rem(wid, segs_per_batch)
    base = wid * b_per_w

    pltpu.sync_copy(ids_hbm.at[batch, pl.ds(seg * b_per_w, b_per_w)], idx_v)

    def gather(c, buf):
        return pltpu.make_async_copy(
            table_hbm.at[idx_v.at[pl.ds(c * CHUNK, CHUNK)]],
            rows_v.at[buf],
            gsems.at[buf],
        )

    def scatter(c, buf):
        return pltpu.make_async_copy(
            rows_v.at[buf],
            out_hbm.at[pl.ds(base + c * CHUNK, CHUNK)],
            ssems.at[buf],
        )

    for k in range(NBUF):
        gather(k, k).start()

    def body(c, carry):
        buf = lax.rem(c, NBUF)

        @pl.when(c >= 1)
        def _():
            pbuf = lax.rem(c - 1, NBUF)
            scatter(c - 1, pbuf).wait()

            @pl.when(c - 1 + NBUF < n_chunks)
            def _():
                gather(c - 1 + NBUF, pbuf).start()

        gather(c, buf).wait()
        scatter(c, buf).start()
        return carry

    lax.fori_loop(0, n_chunks, body, 0)
    scatter(n_chunks - 1, lax.rem(n_chunks - 1, NBUF)).wait()


def kernel(position_ids, positional_encoding):
    batch, seq = position_ids.shape
    n_rows = batch * seq
    b_per_w = n_rows // NUM_WORKERS
    n_chunks = b_per_w // CHUNK
    segs_per_batch = seq // b_per_w

    ids = position_ids.astype(jnp.int32)

    mesh = plsc.VectorSubcoreMesh(core_axis_name="c", subcore_axis_name="s")
    body = functools.partial(_gather_body, b_per_w, n_chunks, segs_per_batch)
    out = pl.kernel(
        body,
        out_type=jax.ShapeDtypeStruct((n_rows, D_MODEL), jnp.float32),
        mesh=mesh,
        scratch_types=[
            pltpu.VMEM((b_per_w,), jnp.int32),
            pltpu.VMEM((NBUF, CHUNK, D_MODEL), jnp.float32),
            pltpu.SemaphoreType.DMA((NBUF,)),
            pltpu.SemaphoreType.DMA((NBUF,)),
        ],
    )(ids, positional_encoding)
    return out.reshape(batch, seq, D_MODEL)
